# Initial kernel scaffold; baseline (speedup 1.0000x reference)
import jax
import jax.numpy as jnp
from jax.experimental import pallas as pl


def kernel(x1, edge_index1, batch1, x2, edge_index2, batch2, gW1_0, gb1_0, gW1_1, gb1_1, gW2_0, gb2_0, gW2_1, gb2_1, mW0, mb0, mW1, mb1, mW2, mb2, mW3, mb3):
    raise NotImplementedError("write your pallas kernel here")


# speedup vs baseline: 10.1363x; 10.1363x over previous
import functools

import jax
import jax.numpy as jnp
from jax import lax
from jax.experimental import pallas as pl
from jax.experimental.pallas import tpu as pltpu
from jax.experimental.pallas import tpu_sc as plsc

_N = 10000
_E = 320000
_D = 128
_NG = 64
_NC = 2
_NS = 16
_CH = 80
_ET = _E // _NS
_NCHUNK = _ET // _CH
_RCH = 80
_NROWCH = _N // _RCH
_ROW_ITERS = -(-_NROWCH // _NS)

_HIGHEST = lax.Precision.HIGHEST

_sc_mesh = plsc.VectorSubcoreMesh(core_axis_name="c", subcore_axis_name="s")


def _sc_counts(dst, bat, ones_rows, zeros_rows):
    @functools.partial(
        pl.kernel,
        out_type=[
            jax.ShapeDtypeStruct((_NC * _N, _D), jnp.float32),
            jax.ShapeDtypeStruct((_NC * _NG, _D), jnp.float32),
        ],
        mesh=_sc_mesh,
        scratch_types=[
            pltpu.VMEM((_CH,), jnp.int32),
            pltpu.VMEM((_CH,), jnp.int32),
            pltpu.VMEM((_CH, _D), jnp.float32),
            pltpu.VMEM((_RCH, _D), jnp.float32),
            pltpu.VMEM_SHARED((_N, _D), jnp.float32),
            pltpu.VMEM_SHARED((_NG, _D), jnp.float32),
        ],
    )
    def k(dst_hbm, bat_hbm, ones_hbm, zer_hbm, cnt_hbm, bcnt_hbm,
          eidx, bidx, ones_v, zer_v, cnt_s, bcnt_s):
        c = lax.axis_index("c")
        t = lax.axis_index("s")
        pltpu.sync_copy(ones_hbm, ones_v)
        pltpu.sync_copy(zer_hbm, zer_v)

        @pl.loop(0, _ROW_ITERS)
        def _(i):
            ch = i * _NS + t

            @pl.when(ch < _NROWCH)
            def _():
                pltpu.sync_copy(zer_v, cnt_s.at[pl.ds(ch * _RCH, _RCH)])

        @pl.when(t == 0)
        def _():
            pltpu.sync_copy(zer_v.at[pl.ds(0, _NG)], bcnt_s)

        plsc.subcore_barrier()

        base_e = c * _E + t * _ET

        @pl.loop(0, _NCHUNK)
        def _(i):
            pltpu.sync_copy(dst_hbm.at[pl.ds(base_e + i * _CH, _CH)], eidx)
            pltpu.sync_copy(ones_v, cnt_s.at[eidx], add=True)

        @pl.loop(0, _ROW_ITERS)
        def _(i):
            ch = i * _NS + t

            @pl.when(ch < _NROWCH)
            def _():
                pltpu.sync_copy(bat_hbm.at[pl.ds(c * _N + ch * _CH, _CH)], bidx)
                pltpu.sync_copy(ones_v, bcnt_s.at[bidx], add=True)

        plsc.subcore_barrier()

        @pl.loop(0, _ROW_ITERS)
        def _(i):
            ch = i * _NS + t

            @pl.when(ch < _NROWCH)
            def _():
                pltpu.sync_copy(cnt_s.at[pl.ds(ch * _RCH, _RCH)],
                                cnt_hbm.at[pl.ds(c * _N + ch * _RCH, _RCH)])

        @pl.when(t == 0)
        def _():
            pltpu.sync_copy(bcnt_s, bcnt_hbm.at[pl.ds(c * _NG, _NG)])

    return k(dst, bat, ones_rows, zeros_rows)


def _sc_conv(xs_flat, src_off, dst, zeros_rows):
    @functools.partial(
        pl.kernel,
        out_type=jax.ShapeDtypeStruct((_NC * _N, _D), jnp.float32),
        mesh=_sc_mesh,
        scratch_types=[
            pltpu.VMEM((_CH,), jnp.int32),
            pltpu.VMEM((_CH,), jnp.int32),
            pltpu.VMEM((_CH, _D), jnp.float32),
            pltpu.VMEM((_RCH, _D), jnp.float32),
            pltpu.VMEM_SHARED((_N, _D), jnp.float32),
        ],
    )
    def k(xs_hbm, src_hbm, dst_hbm, zer_hbm, acc_hbm,
          sidx, didx, rows_v, zer_v, acc_s):
        c = lax.axis_index("c")
        t = lax.axis_index("s")
        pltpu.sync_copy(zer_hbm, zer_v)

        @pl.loop(0, _ROW_ITERS)
        def _(i):
            ch = i * _NS + t

            @pl.when(ch < _NROWCH)
            def _():
                pltpu.sync_copy(zer_v, acc_s.at[pl.ds(ch * _RCH, _RCH)])

        plsc.subcore_barrier()

        base_e = c * _E + t * _ET

        @pl.loop(0, _NCHUNK)
        def _(i):
            pltpu.sync_copy(src_hbm.at[pl.ds(base_e + i * _CH, _CH)], sidx)
            pltpu.sync_copy(dst_hbm.at[pl.ds(base_e + i * _CH, _CH)], didx)
            pltpu.sync_copy(xs_hbm.at[sidx], rows_v)
            pltpu.sync_copy(rows_v, acc_s.at[didx], add=True)

        plsc.subcore_barrier()

        @pl.loop(0, _ROW_ITERS)
        def _(i):
            ch = i * _NS + t

            @pl.when(ch < _NROWCH)
            def _():
                pltpu.sync_copy(acc_s.at[pl.ds(ch * _RCH, _RCH)],
                                acc_hbm.at[pl.ds(c * _N + ch * _RCH, _RCH)])

    return k(xs_flat, src_off, dst, zeros_rows)


_BLK = 2000
_NBLK = _NC * _N // _BLK
_TBLK = _N // _BLK

_row_spec = lambda w: pl.BlockSpec((_BLK, w), lambda i: (i, 0))
_pair_spec2 = pl.BlockSpec((1, 1, _D), lambda i: (i // _TBLK, 0, 0))
_pair_spec3 = pl.BlockSpec((1, _D, _D), lambda i: (i // _TBLK, 0, 0))


def _tc_matmul(x_flat, w_pair):
    def body(x_ref, w_ref, o_ref):
        o_ref[...] = jnp.dot(x_ref[...], w_ref[0].T,
                             preferred_element_type=jnp.float32,
                             precision=_HIGHEST)

    return pl.pallas_call(
        body,
        grid=(_NBLK,),
        in_specs=[_row_spec(_D), _pair_spec3],
        out_specs=_row_spec(_D),
        out_shape=jax.ShapeDtypeStruct((_NC * _N, _D), jnp.float32),
    )(x_flat, w_pair)


def _tc_scale(cnt, h_flat):
    def body(cnt_ref, h_ref, xs_ref, dis_ref):
        dis = lax.rsqrt(cnt_ref[:, 0:1] + 1.0)
        dis_ref[...] = dis
        xs_ref[...] = h_ref[...] * dis

    return pl.pallas_call(
        body,
        grid=(_NBLK,),
        in_specs=[_row_spec(_D), _row_spec(_D)],
        out_specs=[_row_spec(_D), _row_spec(1)],
        out_shape=[
            jax.ShapeDtypeStruct((_NC * _N, _D), jnp.float32),
            jax.ShapeDtypeStruct((_NC * _N, 1), jnp.float32),
        ],
    )(cnt, h_flat)


def _tc_layer(acc_flat, xs_flat, dis, b_pair, w_pair):
    def body(acc_ref, xs_ref, dis_ref, b_ref, w_ref, o_ref):
        d = dis_ref[...]
        o = jax.nn.relu(d * (acc_ref[...] + xs_ref[...]) + b_ref[0])
        h2 = jnp.dot(o, w_ref[0].T, preferred_element_type=jnp.float32,
                     precision=_HIGHEST)
        o_ref[...] = d * h2

    return pl.pallas_call(
        body,
        grid=(_NBLK,),
        in_specs=[_row_spec(_D), _row_spec(_D), _row_spec(1),
                  _pair_spec2, _pair_spec3],
        out_specs=_row_spec(_D),
        out_shape=jax.ShapeDtypeStruct((_NC * _N, _D), jnp.float32),
    )(acc_flat, xs_flat, dis, b_pair, w_pair)


def _tc_pool(acc_flat, xs_flat, dis, b_pair, batf):
    def body(acc_ref, xs_ref, dis_ref, b_ref, bat_ref, s_ref):
        i = pl.program_id(0)
        d = dis_ref[...]
        o = jax.nn.relu(d * (acc_ref[...] + xs_ref[...]) + b_ref[0])
        seg = lax.broadcasted_iota(jnp.int32, (_NG, 1), 0).astype(jnp.float32)
        mask = (bat_ref[0] == seg).astype(jnp.float32)
        s = jnp.dot(mask, o, preferred_element_type=jnp.float32,
                    precision=_HIGHEST)

        @pl.when(i % _TBLK == 0)
        def _():
            s_ref[...] = jnp.zeros_like(s_ref)

        s_ref[0] += s

    return pl.pallas_call(
        body,
        grid=(_NBLK,),
        in_specs=[_row_spec(_D), _row_spec(_D), _row_spec(1), _pair_spec2,
                  pl.BlockSpec((1, 1, _BLK), lambda i: (i, 0, 0))],
        out_specs=pl.BlockSpec((1, _NG, _D), lambda i: (i // _TBLK, 0, 0)),
        out_shape=jax.ShapeDtypeStruct((_NC, _NG, _D), jnp.float32),
    )(acc_flat, xs_flat, dis, b_pair, batf)


def _tc_head(spool, bcnt, mW0, mb0, mW1, mb1, mW2, mb2, mW3, mb3):
    def body(s_ref, bcnt_ref, w0_ref, b0_ref, w1_ref, b1_ref, w2_ref, b2_ref,
             w3_ref, b3_ref, o_ref):
        g1 = s_ref[0] / jnp.maximum(bcnt_ref[0:_NG, 0:1], 1.0)
        g2 = s_ref[1] / jnp.maximum(bcnt_ref[_NG:2 * _NG, 0:1], 1.0)
        z = jnp.concatenate([g1, g2], axis=1)
        for w_ref, bias_ref, act in ((w0_ref, b0_ref, True), (w1_ref, b1_ref, True),
                                     (w2_ref, b2_ref, True), (w3_ref, b3_ref, False)):
            z = jnp.dot(z, w_ref[...].T, preferred_element_type=jnp.float32,
                        precision=_HIGHEST) + bias_ref[...][None, :]
            if act:
                z = jax.nn.relu(z)
        o_ref[...] = z

    return pl.pallas_call(
        body,
        out_shape=jax.ShapeDtypeStruct((_NG, 4), jnp.float32),
    )(spool, bcnt, mW0, mb0, mW1, mb1, mW2, mb2, mW3, mb3)


def kernel(x1, edge_index1, batch1, x2, edge_index2, batch2,
           gW1_0, gb1_0, gW1_1, gb1_1, gW2_0, gb2_0, gW2_1, gb2_1,
           mW0, mb0, mW1, mb1, mW2, mb2, mW3, mb3):
    src = jnp.concatenate([edge_index1[0], edge_index2[0] + _N])
    dst = jnp.concatenate([edge_index1[1], edge_index2[1]])
    bat = jnp.concatenate([batch1, batch2])
    batf = bat.astype(jnp.float32).reshape(_NBLK, 1, _BLK)
    x_flat = jnp.concatenate([x1, x2])

    ones_rows = jnp.ones((_CH, _D), jnp.float32)
    zerosD = jnp.zeros((_RCH, _D), jnp.float32)

    w0 = jnp.stack([gW1_0, gW2_0])
    w1 = jnp.stack([gW1_1, gW2_1])
    b0 = jnp.stack([gb1_0, gb2_0])[:, None, :]
    b1 = jnp.stack([gb1_1, gb2_1])[:, None, :]

    cnt, bcnt = _sc_counts(dst, bat, ones_rows, zerosD)
    h = _tc_matmul(x_flat, w0)
    xs, dis = _tc_scale(cnt, h)
    acc1 = _sc_conv(xs, src, dst, zerosD)
    xs2 = _tc_layer(acc1, xs, dis, b0, w1)
    acc2 = _sc_conv(xs2, src, dst, zerosD)
    spool = _tc_pool(acc2, xs2, dis, b1, batf)
    return _tc_head(spool, bcnt, mW0, mb0, mW1, mb1, mW2, mb2, mW3, mb3)


# speedup vs baseline: 22.4851x; 2.2183x over previous
import functools

import jax
import jax.numpy as jnp
from jax import lax
from jax.experimental import pallas as pl
from jax.experimental.pallas import tpu as pltpu
from jax.experimental.pallas import tpu_sc as plsc

_N = 10000
_E = 320000
_D = 128
_NG = 64
_NC = 2
_NS = 16
_CH = 80
_ET = _E // _NS
_NCHUNK = _ET // _CH
_RCH = 80
_NROWCH = _N // _RCH
_ROW_ITERS = -(-_NROWCH // _NS)

_HIGHEST = lax.Precision.HIGHEST

_sc_mesh = plsc.VectorSubcoreMesh(core_axis_name="c", subcore_axis_name="s")


def _sc_counts(dst, bat, ones_rows, zeros_rows):
    @functools.partial(
        pl.kernel,
        out_type=[
            jax.ShapeDtypeStruct((_NC * _N, _D), jnp.float32),
            jax.ShapeDtypeStruct((_NC * _NG, _D), jnp.float32),
        ],
        mesh=_sc_mesh,
        scratch_types=[
            pltpu.VMEM((_CH,), jnp.int32),
            pltpu.VMEM((_CH,), jnp.int32),
            pltpu.VMEM((_CH, _D), jnp.float32),
            pltpu.VMEM((_RCH, _D), jnp.float32),
            pltpu.VMEM_SHARED((_N, _D), jnp.float32),
            pltpu.VMEM_SHARED((_NG, _D), jnp.float32),
            pltpu.SemaphoreType.DMA,
            pltpu.SemaphoreType.DMA,
        ],
    )
    def k(dst_hbm, bat_hbm, ones_hbm, zer_hbm, cnt_hbm, bcnt_hbm,
          eidx0, eidx1, ones_v, zer_v, cnt_s, bcnt_s, semi0, semi1):
        c = lax.axis_index("c")
        t = lax.axis_index("s")
        pltpu.sync_copy(ones_hbm, ones_v)
        pltpu.sync_copy(zer_hbm, zer_v)

        @pl.loop(0, _ROW_ITERS)
        def _(i):
            ch = i * _NS + t

            @pl.when(ch < _NROWCH)
            def _():
                pltpu.sync_copy(zer_v, cnt_s.at[pl.ds(ch * _RCH, _RCH)])

        @pl.when(t == 0)
        def _():
            pltpu.sync_copy(zer_v.at[pl.ds(0, _NG)], bcnt_s)

        plsc.subcore_barrier()

        base_e = c * _E + t * _ET

        def start_idx(ch_i, ei, sem):
            pltpu.async_copy(dst_hbm.at[pl.ds(base_e + ch_i * _CH, _CH)], ei, sem)

        def wait_idx(ei, sem):
            pltpu.make_async_copy(dst_hbm.at[pl.ds(0, _CH)], ei, sem).wait()

        start_idx(0, eidx0, semi0)
        start_idx(1, eidx1, semi1)

        @pl.loop(0, _NCHUNK // 2)
        def _(g):
            c0 = 2 * g
            wait_idx(eidx0, semi0)
            pltpu.sync_copy(ones_v, cnt_s.at[eidx0], add=True)

            @pl.when(c0 + 2 < _NCHUNK)
            def _():
                start_idx(c0 + 2, eidx0, semi0)

            wait_idx(eidx1, semi1)
            pltpu.sync_copy(ones_v, cnt_s.at[eidx1], add=True)

            @pl.when(c0 + 3 < _NCHUNK)
            def _():
                start_idx(c0 + 3, eidx1, semi1)

        @pl.loop(0, _ROW_ITERS)
        def _(i):
            ch = i * _NS + t

            @pl.when(ch < _NROWCH)
            def _():
                pltpu.sync_copy(bat_hbm.at[pl.ds(c * _N + ch * _CH, _CH)], eidx0)
                pltpu.sync_copy(ones_v, bcnt_s.at[eidx0], add=True)

        plsc.subcore_barrier()

        @pl.loop(0, _ROW_ITERS)
        def _(i):
            ch = i * _NS + t

            @pl.when(ch < _NROWCH)
            def _():
                pltpu.sync_copy(cnt_s.at[pl.ds(ch * _RCH, _RCH)],
                                cnt_hbm.at[pl.ds(c * _N + ch * _RCH, _RCH)])

        @pl.when(t == 0)
        def _():
            pltpu.sync_copy(bcnt_s, bcnt_hbm.at[pl.ds(c * _NG, _NG)])

    return k(dst, bat, ones_rows, zeros_rows)


def _sc_conv(xs_flat, src_off, dst, zeros_rows):
    @functools.partial(
        pl.kernel,
        out_type=jax.ShapeDtypeStruct((_NC * _N, _D), jnp.float32),
        mesh=_sc_mesh,
        scratch_types=[
            pltpu.VMEM((_CH,), jnp.int32),
            pltpu.VMEM((_CH,), jnp.int32),
            pltpu.VMEM((_CH,), jnp.int32),
            pltpu.VMEM((_CH,), jnp.int32),
            pltpu.VMEM((_CH, _D), jnp.float32),
            pltpu.VMEM((_CH, _D), jnp.float32),
            pltpu.VMEM((_RCH, _D), jnp.float32),
            pltpu.VMEM_SHARED((_N, _D), jnp.float32),
            pltpu.SemaphoreType.DMA,
            pltpu.SemaphoreType.DMA,
            pltpu.SemaphoreType.DMA,
            pltpu.SemaphoreType.DMA,
        ],
    )
    def k(xs_hbm, src_hbm, dst_hbm, zer_hbm, acc_hbm,
          sidx0, didx0, sidx1, didx1, rows0, rows1, zer_v, acc_s,
          semi0, semi1, semg0, semg1):
        c = lax.axis_index("c")
        t = lax.axis_index("s")
        pltpu.sync_copy(zer_hbm, zer_v)

        @pl.loop(0, _ROW_ITERS)
        def _(i):
            ch = i * _NS + t

            @pl.when(ch < _NROWCH)
            def _():
                pltpu.sync_copy(zer_v, acc_s.at[pl.ds(ch * _RCH, _RCH)])

        plsc.subcore_barrier()

        base_e = c * _E + t * _ET

        def start_idx(ch_i, si, di, sem):
            off = base_e + ch_i * _CH
            pltpu.async_copy(src_hbm.at[pl.ds(off, _CH)], si, sem)
            pltpu.async_copy(dst_hbm.at[pl.ds(off, _CH)], di, sem)

        def wait_idx(si, di, sem):
            pltpu.make_async_copy(src_hbm.at[pl.ds(0, _CH)], si, sem).wait()
            pltpu.make_async_copy(dst_hbm.at[pl.ds(0, _CH)], di, sem).wait()

        def start_gather(si, rows, sem):
            pltpu.async_copy(xs_hbm.at[si], rows, sem)

        def wait_gather(si, rows, sem):
            pltpu.make_async_copy(xs_hbm.at[si], rows, sem).wait()

        start_idx(0, sidx0, didx0, semi0)
        start_idx(1, sidx1, didx1, semi1)
        wait_idx(sidx0, didx0, semi0)
        start_gather(sidx0, rows0, semg0)

        @pl.loop(0, _NCHUNK // 2)
        def _(g):
            c0 = 2 * g
            wait_idx(sidx1, didx1, semi1)
            start_gather(sidx1, rows1, semg1)
            wait_gather(sidx0, rows0, semg0)

            @pl.when(c0 + 2 < _NCHUNK)
            def _():
                start_idx(c0 + 2, sidx0, didx0, semi0)

            pltpu.sync_copy(rows0, acc_s.at[didx0], add=True)
            @pl.when(c0 + 2 < _NCHUNK)
            def _():
                wait_idx(sidx0, didx0, semi0)
                start_gather(sidx0, rows0, semg0)

            wait_gather(sidx1, rows1, semg1)

            @pl.when(c0 + 3 < _NCHUNK)
            def _():
                start_idx(c0 + 3, sidx1, didx1, semi1)

            pltpu.sync_copy(rows1, acc_s.at[didx1], add=True)

        plsc.subcore_barrier()

        @pl.loop(0, _ROW_ITERS)
        def _(i):
            ch = i * _NS + t

            @pl.when(ch < _NROWCH)
            def _():
                pltpu.sync_copy(acc_s.at[pl.ds(ch * _RCH, _RCH)],
                                acc_hbm.at[pl.ds(c * _N + ch * _RCH, _RCH)])

    return k(xs_flat, src_off, dst, zeros_rows)


_BLK = 2000
_NBLK = _NC * _N // _BLK
_TBLK = _N // _BLK

_row_spec = lambda w: pl.BlockSpec((_BLK, w), lambda i: (i, 0))
_pair_spec2 = pl.BlockSpec((1, 1, _D), lambda i: (i // _TBLK, 0, 0))
_pair_spec3 = pl.BlockSpec((1, _D, _D), lambda i: (i // _TBLK, 0, 0))


def _tc_matmul(x_flat, w_pair):
    def body(x_ref, w_ref, o_ref):
        o_ref[...] = jnp.dot(x_ref[...], w_ref[0].T,
                             preferred_element_type=jnp.float32,
                             precision=_HIGHEST)

    return pl.pallas_call(
        body,
        grid=(_NBLK,),
        in_specs=[_row_spec(_D), _pair_spec3],
        out_specs=_row_spec(_D),
        out_shape=jax.ShapeDtypeStruct((_NC * _N, _D), jnp.float32),
    )(x_flat, w_pair)


def _tc_scale(cnt, h_flat):
    def body(cnt_ref, h_ref, xs_ref, dis_ref):
        dis = lax.rsqrt(cnt_ref[:, 0:1] + 1.0)
        dis_ref[...] = dis
        xs_ref[...] = h_ref[...] * dis

    return pl.pallas_call(
        body,
        grid=(_NBLK,),
        in_specs=[_row_spec(_D), _row_spec(_D)],
        out_specs=[_row_spec(_D), _row_spec(1)],
        out_shape=[
            jax.ShapeDtypeStruct((_NC * _N, _D), jnp.float32),
            jax.ShapeDtypeStruct((_NC * _N, 1), jnp.float32),
        ],
    )(cnt, h_flat)


def _tc_layer(acc_flat, xs_flat, dis, b_pair, w_pair):
    def body(acc_ref, xs_ref, dis_ref, b_ref, w_ref, o_ref):
        d = dis_ref[...]
        o = jax.nn.relu(d * (acc_ref[...] + xs_ref[...]) + b_ref[0])
        h2 = jnp.dot(o, w_ref[0].T, preferred_element_type=jnp.float32,
                     precision=_HIGHEST)
        o_ref[...] = d * h2

    return pl.pallas_call(
        body,
        grid=(_NBLK,),
        in_specs=[_row_spec(_D), _row_spec(_D), _row_spec(1),
                  _pair_spec2, _pair_spec3],
        out_specs=_row_spec(_D),
        out_shape=jax.ShapeDtypeStruct((_NC * _N, _D), jnp.float32),
    )(acc_flat, xs_flat, dis, b_pair, w_pair)


def _tc_pool(acc_flat, xs_flat, dis, b_pair, batf):
    def body(acc_ref, xs_ref, dis_ref, b_ref, bat_ref, s_ref):
        i = pl.program_id(0)
        d = dis_ref[...]
        o = jax.nn.relu(d * (acc_ref[...] + xs_ref[...]) + b_ref[0])
        seg = lax.broadcasted_iota(jnp.int32, (_NG, 1), 0).astype(jnp.float32)
        mask = (bat_ref[0] == seg).astype(jnp.float32)
        s = jnp.dot(mask, o, preferred_element_type=jnp.float32,
                    precision=_HIGHEST)

        @pl.when(i % _TBLK == 0)
        def _():
            s_ref[...] = jnp.zeros_like(s_ref)

        s_ref[0] += s

    return pl.pallas_call(
        body,
        grid=(_NBLK,),
        in_specs=[_row_spec(_D), _row_spec(_D), _row_spec(1), _pair_spec2,
                  pl.BlockSpec((1, 1, _BLK), lambda i: (i, 0, 0))],
        out_specs=pl.BlockSpec((1, _NG, _D), lambda i: (i // _TBLK, 0, 0)),
        out_shape=jax.ShapeDtypeStruct((_NC, _NG, _D), jnp.float32),
    )(acc_flat, xs_flat, dis, b_pair, batf)


def _tc_head(spool, bcnt, mW0, mb0, mW1, mb1, mW2, mb2, mW3, mb3):
    def body(s_ref, bcnt_ref, w0_ref, b0_ref, w1_ref, b1_ref, w2_ref, b2_ref,
             w3_ref, b3_ref, o_ref):
        g1 = s_ref[0] / jnp.maximum(bcnt_ref[0:_NG, 0:1], 1.0)
        g2 = s_ref[1] / jnp.maximum(bcnt_ref[_NG:2 * _NG, 0:1], 1.0)
        z = jnp.concatenate([g1, g2], axis=1)
        for w_ref, bias_ref, act in ((w0_ref, b0_ref, True), (w1_ref, b1_ref, True),
                                     (w2_ref, b2_ref, True), (w3_ref, b3_ref, False)):
            z = jnp.dot(z, w_ref[...].T, preferred_element_type=jnp.float32,
                        precision=_HIGHEST) + bias_ref[...][None, :]
            if act:
                z = jax.nn.relu(z)
        o_ref[...] = z

    return pl.pallas_call(
        body,
        out_shape=jax.ShapeDtypeStruct((_NG, 4), jnp.float32),
    )(spool, bcnt, mW0, mb0, mW1, mb1, mW2, mb2, mW3, mb3)


def kernel(x1, edge_index1, batch1, x2, edge_index2, batch2,
           gW1_0, gb1_0, gW1_1, gb1_1, gW2_0, gb2_0, gW2_1, gb2_1,
           mW0, mb0, mW1, mb1, mW2, mb2, mW3, mb3):
    src = jnp.concatenate([edge_index1[0], edge_index2[0] + _N])
    dst = jnp.concatenate([edge_index1[1], edge_index2[1]])
    bat = jnp.concatenate([batch1, batch2])
    batf = bat.astype(jnp.float32).reshape(_NBLK, 1, _BLK)
    x_flat = jnp.concatenate([x1, x2])

    ones_rows = jnp.ones((_CH, _D), jnp.float32)
    zerosD = jnp.zeros((_RCH, _D), jnp.float32)

    w0 = jnp.stack([gW1_0, gW2_0])
    w1 = jnp.stack([gW1_1, gW2_1])
    b0 = jnp.stack([gb1_0, gb2_0])[:, None, :]
    b1 = jnp.stack([gb1_1, gb2_1])[:, None, :]

    cnt, bcnt = _sc_counts(dst, bat, ones_rows, zerosD)
    h = _tc_matmul(x_flat, w0)
    xs, dis = _tc_scale(cnt, h)
    acc1 = _sc_conv(xs, src, dst, zerosD)
    xs2 = _tc_layer(acc1, xs, dis, b0, w1)
    acc2 = _sc_conv(xs2, src, dst, zerosD)
    spool = _tc_pool(acc2, xs2, dis, b1, batf)
    return _tc_head(spool, bcnt, mW0, mb0, mW1, mb1, mW2, mb2, mW3, mb3)


# speedup vs baseline: 22.6222x; 1.0061x over previous
import functools

import jax
import jax.numpy as jnp
from jax import lax
from jax.experimental import pallas as pl
from jax.experimental.pallas import tpu as pltpu
from jax.experimental.pallas import tpu_sc as plsc

_N = 10000
_E = 320000
_D = 128
_NG = 64
_NC = 2
_NS = 16
_CH = 80
_ET = _E // _NS
_NCHUNK = _ET // _CH
_RCH = 80
_NROWCH = _N // _RCH
_ROW_ITERS = -(-_NROWCH // _NS)

_HIGHEST = lax.Precision.HIGHEST


def _dot_bf16(a, b):
    return jnp.dot(a.astype(jnp.bfloat16), b.astype(jnp.bfloat16),
                   preferred_element_type=jnp.float32)

_sc_mesh = plsc.VectorSubcoreMesh(core_axis_name="c", subcore_axis_name="s")


def _sc_counts(dst, bat, ones_rows, zeros_rows):
    @functools.partial(
        pl.kernel,
        out_type=[
            jax.ShapeDtypeStruct((_NC * _N, _D), jnp.float32),
            jax.ShapeDtypeStruct((_NC * _NG, _D), jnp.float32),
        ],
        mesh=_sc_mesh,
        scratch_types=[
            pltpu.VMEM((_CH,), jnp.int32),
            pltpu.VMEM((_CH,), jnp.int32),
            pltpu.VMEM((_CH, _D), jnp.float32),
            pltpu.VMEM((_RCH, _D), jnp.float32),
            pltpu.VMEM_SHARED((_N, _D), jnp.float32),
            pltpu.VMEM_SHARED((_NG, _D), jnp.float32),
            pltpu.SemaphoreType.DMA,
            pltpu.SemaphoreType.DMA,
        ],
    )
    def k(dst_hbm, bat_hbm, ones_hbm, zer_hbm, cnt_hbm, bcnt_hbm,
          eidx0, eidx1, ones_v, zer_v, cnt_s, bcnt_s, semi0, semi1):
        c = lax.axis_index("c")
        t = lax.axis_index("s")
        pltpu.sync_copy(ones_hbm, ones_v)
        pltpu.sync_copy(zer_hbm, zer_v)

        @pl.loop(0, _ROW_ITERS)
        def _(i):
            ch = i * _NS + t

            @pl.when(ch < _NROWCH)
            def _():
                pltpu.sync_copy(zer_v, cnt_s.at[pl.ds(ch * _RCH, _RCH)])

        @pl.when(t == 0)
        def _():
            pltpu.sync_copy(zer_v.at[pl.ds(0, _NG)], bcnt_s)

        plsc.subcore_barrier()

        base_e = c * _E + t * _ET

        def start_idx(ch_i, ei, sem):
            pltpu.async_copy(dst_hbm.at[pl.ds(base_e + ch_i * _CH, _CH)], ei, sem)

        def wait_idx(ei, sem):
            pltpu.make_async_copy(dst_hbm.at[pl.ds(0, _CH)], ei, sem).wait()

        start_idx(0, eidx0, semi0)
        start_idx(1, eidx1, semi1)

        @pl.loop(0, _NCHUNK // 2)
        def _(g):
            c0 = 2 * g
            wait_idx(eidx0, semi0)
            pltpu.sync_copy(ones_v, cnt_s.at[eidx0], add=True)

            @pl.when(c0 + 2 < _NCHUNK)
            def _():
                start_idx(c0 + 2, eidx0, semi0)

            wait_idx(eidx1, semi1)
            pltpu.sync_copy(ones_v, cnt_s.at[eidx1], add=True)

            @pl.when(c0 + 3 < _NCHUNK)
            def _():
                start_idx(c0 + 3, eidx1, semi1)

        @pl.loop(0, _ROW_ITERS)
        def _(i):
            ch = i * _NS + t

            @pl.when(ch < _NROWCH)
            def _():
                pltpu.sync_copy(bat_hbm.at[pl.ds(c * _N + ch * _CH, _CH)], eidx0)
                pltpu.sync_copy(ones_v, bcnt_s.at[eidx0], add=True)

        plsc.subcore_barrier()

        @pl.loop(0, _ROW_ITERS)
        def _(i):
            ch = i * _NS + t

            @pl.when(ch < _NROWCH)
            def _():
                pltpu.sync_copy(cnt_s.at[pl.ds(ch * _RCH, _RCH)],
                                cnt_hbm.at[pl.ds(c * _N + ch * _RCH, _RCH)])

        @pl.when(t == 0)
        def _():
            pltpu.sync_copy(bcnt_s, bcnt_hbm.at[pl.ds(c * _NG, _NG)])

    return k(dst, bat, ones_rows, zeros_rows)


def _sc_conv(xs_flat, src_off, dst, zeros_rows):
    @functools.partial(
        pl.kernel,
        out_type=jax.ShapeDtypeStruct((_NC * _N, _D), jnp.float32),
        mesh=_sc_mesh,
        scratch_types=[
            pltpu.VMEM((_CH,), jnp.int32),
            pltpu.VMEM((_CH,), jnp.int32),
            pltpu.VMEM((_CH,), jnp.int32),
            pltpu.VMEM((_CH,), jnp.int32),
            pltpu.VMEM((_CH, _D), jnp.float32),
            pltpu.VMEM((_CH, _D), jnp.float32),
            pltpu.VMEM((_RCH, _D), jnp.float32),
            pltpu.VMEM_SHARED((_N, _D), jnp.float32),
            pltpu.SemaphoreType.DMA,
            pltpu.SemaphoreType.DMA,
            pltpu.SemaphoreType.DMA,
            pltpu.SemaphoreType.DMA,
        ],
    )
    def k(xs_hbm, src_hbm, dst_hbm, zer_hbm, acc_hbm,
          sidx0, didx0, sidx1, didx1, rows0, rows1, zer_v, acc_s,
          semi0, semi1, semg0, semg1):
        c = lax.axis_index("c")
        t = lax.axis_index("s")
        pltpu.sync_copy(zer_hbm, zer_v)

        @pl.loop(0, _ROW_ITERS)
        def _(i):
            ch = i * _NS + t

            @pl.when(ch < _NROWCH)
            def _():
                pltpu.sync_copy(zer_v, acc_s.at[pl.ds(ch * _RCH, _RCH)])

        plsc.subcore_barrier()

        base_e = c * _E + t * _ET

        def start_idx(ch_i, si, di, sem):
            off = base_e + ch_i * _CH
            pltpu.async_copy(src_hbm.at[pl.ds(off, _CH)], si, sem)
            pltpu.async_copy(dst_hbm.at[pl.ds(off, _CH)], di, sem)

        def wait_idx(si, di, sem):
            pltpu.make_async_copy(src_hbm.at[pl.ds(0, _CH)], si, sem).wait()
            pltpu.make_async_copy(dst_hbm.at[pl.ds(0, _CH)], di, sem).wait()

        def start_gather(si, rows, sem):
            pltpu.async_copy(xs_hbm.at[si], rows, sem)

        def wait_gather(si, rows, sem):
            pltpu.make_async_copy(xs_hbm.at[si], rows, sem).wait()

        start_idx(0, sidx0, didx0, semi0)
        start_idx(1, sidx1, didx1, semi1)
        wait_idx(sidx0, didx0, semi0)
        start_gather(sidx0, rows0, semg0)

        @pl.loop(0, _NCHUNK // 2)
        def _(g):
            c0 = 2 * g
            wait_idx(sidx1, didx1, semi1)
            start_gather(sidx1, rows1, semg1)
            wait_gather(sidx0, rows0, semg0)

            @pl.when(c0 + 2 < _NCHUNK)
            def _():
                start_idx(c0 + 2, sidx0, didx0, semi0)

            pltpu.sync_copy(rows0, acc_s.at[didx0], add=True)
            @pl.when(c0 + 2 < _NCHUNK)
            def _():
                wait_idx(sidx0, didx0, semi0)
                start_gather(sidx0, rows0, semg0)

            wait_gather(sidx1, rows1, semg1)

            @pl.when(c0 + 3 < _NCHUNK)
            def _():
                start_idx(c0 + 3, sidx1, didx1, semi1)

            pltpu.sync_copy(rows1, acc_s.at[didx1], add=True)

        plsc.subcore_barrier()

        @pl.loop(0, _ROW_ITERS)
        def _(i):
            ch = i * _NS + t

            @pl.when(ch < _NROWCH)
            def _():
                pltpu.sync_copy(acc_s.at[pl.ds(ch * _RCH, _RCH)],
                                acc_hbm.at[pl.ds(c * _N + ch * _RCH, _RCH)])

    return k(xs_flat, src_off, dst, zeros_rows)


_BLK = 2000
_NBLK = _NC * _N // _BLK
_TBLK = _N // _BLK

_row_spec = lambda w: pl.BlockSpec((_BLK, w), lambda i: (i, 0))
_pair_spec2 = pl.BlockSpec((1, 1, _D), lambda i: (i // _TBLK, 0, 0))
_pair_spec3 = pl.BlockSpec((1, _D, _D), lambda i: (i // _TBLK, 0, 0))


def _tc_matmul(x_flat, w_pair):
    def body(x_ref, w_ref, o_ref):
        o_ref[...] = _dot_bf16(x_ref[...], w_ref[0].T)

    return pl.pallas_call(
        body,
        grid=(_NBLK,),
        in_specs=[_row_spec(_D), _pair_spec3],
        out_specs=_row_spec(_D),
        out_shape=jax.ShapeDtypeStruct((_NC * _N, _D), jnp.float32),
    )(x_flat, w_pair)


def _tc_scale(cnt, h_flat):
    def body(cnt_ref, h_ref, xs_ref, dis_ref):
        dis = lax.rsqrt(cnt_ref[:, 0:1] + 1.0)
        dis_ref[...] = dis
        xs_ref[...] = h_ref[...] * dis

    return pl.pallas_call(
        body,
        grid=(_NBLK,),
        in_specs=[_row_spec(_D), _row_spec(_D)],
        out_specs=[_row_spec(_D), _row_spec(1)],
        out_shape=[
            jax.ShapeDtypeStruct((_NC * _N, _D), jnp.float32),
            jax.ShapeDtypeStruct((_NC * _N, 1), jnp.float32),
        ],
    )(cnt, h_flat)


def _tc_layer(acc_flat, xs_flat, dis, b_pair, w_pair):
    def body(acc_ref, xs_ref, dis_ref, b_ref, w_ref, o_ref):
        d = dis_ref[...]
        o = jax.nn.relu(d * (acc_ref[...] + xs_ref[...]) + b_ref[0])
        h2 = _dot_bf16(o, w_ref[0].T)
        o_ref[...] = d * h2

    return pl.pallas_call(
        body,
        grid=(_NBLK,),
        in_specs=[_row_spec(_D), _row_spec(_D), _row_spec(1),
                  _pair_spec2, _pair_spec3],
        out_specs=_row_spec(_D),
        out_shape=jax.ShapeDtypeStruct((_NC * _N, _D), jnp.float32),
    )(acc_flat, xs_flat, dis, b_pair, w_pair)


def _tc_pool(acc_flat, xs_flat, dis, b_pair, batf):
    def body(acc_ref, xs_ref, dis_ref, b_ref, bat_ref, s_ref):
        i = pl.program_id(0)
        d = dis_ref[...]
        o = jax.nn.relu(d * (acc_ref[...] + xs_ref[...]) + b_ref[0])
        seg = lax.broadcasted_iota(jnp.int32, (_NG, 1), 0).astype(jnp.float32)
        mask = (bat_ref[0] == seg).astype(jnp.float32)
        s = jnp.dot(mask, o, preferred_element_type=jnp.float32,
                    precision=_HIGHEST)

        @pl.when(i % _TBLK == 0)
        def _():
            s_ref[...] = jnp.zeros_like(s_ref)

        s_ref[0] += s

    return pl.pallas_call(
        body,
        grid=(_NBLK,),
        in_specs=[_row_spec(_D), _row_spec(_D), _row_spec(1), _pair_spec2,
                  pl.BlockSpec((1, 1, _BLK), lambda i: (i, 0, 0))],
        out_specs=pl.BlockSpec((1, _NG, _D), lambda i: (i // _TBLK, 0, 0)),
        out_shape=jax.ShapeDtypeStruct((_NC, _NG, _D), jnp.float32),
    )(acc_flat, xs_flat, dis, b_pair, batf)


def _tc_head(spool, bcnt, mW0, mb0, mW1, mb1, mW2, mb2, mW3, mb3):
    def body(s_ref, bcnt_ref, w0_ref, b0_ref, w1_ref, b1_ref, w2_ref, b2_ref,
             w3_ref, b3_ref, o_ref):
        g1 = s_ref[0] / jnp.maximum(bcnt_ref[0:_NG, 0:1], 1.0)
        g2 = s_ref[1] / jnp.maximum(bcnt_ref[_NG:2 * _NG, 0:1], 1.0)
        z = jnp.concatenate([g1, g2], axis=1)
        for w_ref, bias_ref, act in ((w0_ref, b0_ref, True), (w1_ref, b1_ref, True),
                                     (w2_ref, b2_ref, True), (w3_ref, b3_ref, False)):
            z = _dot_bf16(z, w_ref[...].T) + bias_ref[...][None, :]
            if act:
                z = jax.nn.relu(z)
        o_ref[...] = z

    return pl.pallas_call(
        body,
        out_shape=jax.ShapeDtypeStruct((_NG, 4), jnp.float32),
    )(spool, bcnt, mW0, mb0, mW1, mb1, mW2, mb2, mW3, mb3)


def kernel(x1, edge_index1, batch1, x2, edge_index2, batch2,
           gW1_0, gb1_0, gW1_1, gb1_1, gW2_0, gb2_0, gW2_1, gb2_1,
           mW0, mb0, mW1, mb1, mW2, mb2, mW3, mb3):
    src = jnp.concatenate([edge_index1[0], edge_index2[0] + _N])
    dst = jnp.concatenate([edge_index1[1], edge_index2[1]])
    bat = jnp.concatenate([batch1, batch2])
    batf = bat.astype(jnp.float32).reshape(_NBLK, 1, _BLK)
    x_flat = jnp.concatenate([x1, x2])

    ones_rows = jnp.ones((_CH, _D), jnp.float32)
    zerosD = jnp.zeros((_RCH, _D), jnp.float32)

    w0 = jnp.stack([gW1_0, gW2_0])
    w1 = jnp.stack([gW1_1, gW2_1])
    b0 = jnp.stack([gb1_0, gb2_0])[:, None, :]
    b1 = jnp.stack([gb1_1, gb2_1])[:, None, :]

    cnt, bcnt = _sc_counts(dst, bat, ones_rows, zerosD)
    h = _tc_matmul(x_flat, w0)
    xs, dis = _tc_scale(cnt, h)
    acc1 = _sc_conv(xs, src, dst, zerosD)
    xs2 = _tc_layer(acc1, xs, dis, b0, w1)
    acc2 = _sc_conv(xs2, src, dst, zerosD)
    spool = _tc_pool(acc2, xs2, dis, b1, batf)
    return _tc_head(spool, bcnt, mW0, mb0, mW1, mb1, mW2, mb2, mW3, mb3)
